# baseline (speedup 1.0000x reference)
import functools

import jax
import jax.numpy as jnp
from jax import lax
from jax.experimental import pallas as pl
from jax.experimental.pallas import tpu as pltpu
from jax.experimental.pallas import tpu_sc as plsc

B = 16384
NF = 64
D_IMG = 512
NC = 2
NS = 16
NW = NC * NS
BPW = B // NW
CH = 128

_sc_mesh = plsc.VectorSubcoreMesh(core_axis_name="c", subcore_axis_name="s")


@functools.partial(
    pl.kernel,
    mesh=_sc_mesh,
    out_type=[
        jax.ShapeDtypeStruct((B, NF), jnp.float32),
        jax.ShapeDtypeStruct((B,), jnp.float32),
    ],
    scratch_types=[
        pltpu.VMEM((BPW,), jnp.int32),
        pltpu.VMEM((BPW,), jnp.int32),
        pltpu.VMEM((BPW, NF), jnp.float32),
        pltpu.VMEM((BPW,), jnp.float32),
        pltpu.VMEM((BPW,), jnp.float32),
        pltpu.VMEM((BPW,), jnp.float32),
        pltpu.SemaphoreType.DMA,
        pltpu.SemaphoreType.DMA,
    ],
    compiler_params=pltpu.CompilerParams(use_tc_tiling_on_sc=False),
)
def _sc_gather(user_hbm, item_hbm, uf_hbm, ub_hbm, ib_hbm,
               ue_out, bias_out,
               uidx_v, iidx_v, rows_v, ub_v, ib_v, bias_v, sem_r, sem_b):
    wid = lax.axis_index("s") * NC + lax.axis_index("c")
    base = wid * BPW
    pltpu.sync_copy(user_hbm.at[pl.ds(base, BPW)], uidx_v)
    pltpu.sync_copy(item_hbm.at[pl.ds(base, BPW)], iidx_v)
    copies = []
    for j in range(BPW // CH):
        sl = pl.ds(j * CH, CH)
        copies.append(
            pltpu.async_copy(uf_hbm.at[uidx_v.at[sl]], rows_v.at[sl], sem_r))
        copies.append(
            pltpu.async_copy(ub_hbm.at[uidx_v.at[sl]], ub_v.at[sl], sem_b))
        copies.append(
            pltpu.async_copy(ib_hbm.at[iidx_v.at[sl]], ib_v.at[sl], sem_b))
    for c in copies:
        c.wait()
    for j in range(BPW // 16):
        sl = pl.ds(j * 16, 16)
        bias_v[sl] = ub_v[sl] + ib_v[sl]
    pltpu.sync_copy(rows_v, ue_out.at[pl.ds(base, BPW)])
    pltpu.sync_copy(bias_v, bias_out.at[pl.ds(base, BPW)])


BLK = 1024


def _tc_body(img_ref, w_ref, b_ref, ue_ref, bias_ref, out_ref):
    imf = jnp.dot(img_ref[...], w_ref[...],
                  preferred_element_type=jnp.float32) + b_ref[...]
    s = jnp.sum(imf * ue_ref[...], axis=1, keepdims=True)
    out_ref[...] = s + bias_ref[...]


_tc_combine = pl.pallas_call(
    _tc_body,
    grid=(B // BLK,),
    in_specs=[
        pl.BlockSpec((BLK, D_IMG), lambda i: (i, 0)),
        pl.BlockSpec((D_IMG, NF), lambda i: (0, 0)),
        pl.BlockSpec((1, NF), lambda i: (0, 0)),
        pl.BlockSpec((BLK, NF), lambda i: (i, 0)),
        pl.BlockSpec((BLK, 1), lambda i: (i, 0)),
    ],
    out_specs=pl.BlockSpec((BLK, 1), lambda i: (i, 0)),
    out_shape=jax.ShapeDtypeStruct((B, 1), jnp.float32),
)


def kernel(image, user, item, user_factors, user_biases, item_biases,
           W_img, b_img):
    user = user.astype(jnp.int32)
    item = item.astype(jnp.int32)
    ub_flat = user_biases.reshape(-1)
    ib_flat = item_biases.reshape(-1)
    ue, bias = _sc_gather(user, item, user_factors, ub_flat, ib_flat)
    pred = _tc_combine(image, W_img, b_img.reshape(1, NF), ue,
                       bias.reshape(B, 1))
    return pred.reshape(B)
